# baseline (speedup 1.0000x reference)
import functools

import jax
import jax.numpy as jnp
import numpy as np
from jax import lax
from jax.experimental import pallas as pl
from jax.experimental.pallas import tpu as pltpu
from jax.experimental.pallas import tpu_sc as plsc

N_ELEMS = 2 * 128 * 128 * 128
K_TOP = int(N_ELEMS * 10 / 100)
SMOOTH = np.float32(1e-4)
EPSILON = np.float32(1e-5)
MAGN = 0x7FFFFFFF

NC, NS, LANES = 2, 16, 16
NW = NC * NS
EPT = N_ELEMS // NW
CHUNK = 16384
NCHUNKS = EPT // CHUNK

_MESH = plsc.VectorSubcoreMesh(
    core_axis_name="c", subcore_axis_name="s", num_cores=NC, num_subcores=NS)
_SC_PARAMS = pltpu.CompilerParams(needs_layout_passes=False)


def _merge_parts(h_hbm, blkb, mergedb, nbins):
    zeros = jnp.zeros((16,), jnp.int32)

    @plsc.parallel_loop(0, nbins // 16, unroll=8)
    def _(i):
        mergedb[pl.ds(i * 16, 16)] = zeros

    for blk in range(NW // 8):
        pltpu.sync_copy(h_hbm.at[pl.ds(blk * 8 * nbins, 8 * nbins)], blkb)

        @plsc.parallel_loop(0, nbins // 16, unroll=4)
        def _(g):
            acc = mergedb[pl.ds(g * 16, 16)]
            for r in range(8):
                acc = acc + blkb[pl.ds(r * nbins + g * 16, 16)]
            mergedb[pl.ds(g * 16, 16)] = acc


def _scan_merged(mergedb, nbins, kk):
    def body(g, carry):
        b_acc, c_acc, run = carry
        acc = mergedb[pl.ds(g * 16, 16)]
        incl = plsc.cumsum(acc) + run
        mask = incl < kk
        b_acc = b_acc + jnp.sum(jnp.where(mask, 1, 0))
        c_acc = c_acc + jnp.sum(jnp.where(mask, acc, 0))
        run = run + jnp.sum(acc)
        return b_acc, c_acc, run

    b, c, _ = lax.fori_loop(0, nbins // 16, body,
                            (jnp.int32(0), jnp.int32(0), jnp.int32(0)))
    return b, c


def _zero_hist(histb, nwords):
    zeros = jnp.zeros((16,), jnp.int32)

    @plsc.parallel_loop(0, nwords // 16, unroll=8)
    def _(i):
        histb[pl.ds(i * 16, 16)] = zeros


def _collapse(histb, collb, nbins):
    @plsc.parallel_loop(0, nbins // 16, unroll=2)
    def _(g):
        acc = jnp.zeros((16,), jnp.int32)
        for l in range(LANES):
            acc = acc + histb[pl.ds(l * nbins + g * 16, 16)]
        collb[pl.ds(g * 16, 16)] = acc


def _emit_meta(metab, meta_hbm, wid, vals):
    idx16 = lax.iota(jnp.int32, 16)
    v = jnp.zeros((16,), jnp.int32)
    for j, s in enumerate(vals):
        v = jnp.where(idx16 == j, s, v)

    @pl.when(wid == 0)
    def _():
        metab[...] = v
        pltpu.sync_copy(metab, meta_hbm)


def _read_meta(metab, meta_hbm, j):
    idx16 = lax.iota(jnp.int32, 16)
    pltpu.sync_copy(meta_hbm, metab)
    v = metab[...]
    return jnp.sum(jnp.where(idx16 == j, v, 0))


@functools.partial(
    pl.kernel,
    out_type=[jax.ShapeDtypeStruct((N_ELEMS,), jnp.float32),
              jax.ShapeDtypeStruct((NW * 2048,), jnp.int32)],
    mesh=_MESH,
    compiler_params=_SC_PARAMS,
    scratch_types=[
        pltpu.VMEM((2, CHUNK), jnp.float32),
        pltpu.VMEM((2, CHUNK), jnp.float32),
        pltpu.VMEM((LANES * 2048,), jnp.int32),
        pltpu.VMEM((2048,), jnp.int32),
        pltpu.SemaphoreType.DMA,
        pltpu.SemaphoreType.DMA,
        pltpu.SemaphoreType.DMA,
        pltpu.SemaphoreType.DMA,
    ],
)
def _sc_pass1(net_hbm, tar_hbm, u_hbm, h1_hbm,
              netb, tarb, histb, collb, seml0, seml1, semu0, semu1):
    wid = lax.axis_index("s") * NC + lax.axis_index("c")
    _zero_hist(histb, LANES * 2048)
    lane2048 = lax.iota(jnp.int32, 16) * 2048
    ones = jnp.ones((16,), jnp.int32)
    seml = [seml0, seml1]
    semu = [semu0, semu1]

    def issue_load(ci):
        b = ci % 2
        base = wid * EPT + ci * CHUNK
        return (pltpu.async_copy(net_hbm.at[pl.ds(base, CHUNK)],
                                 netb.at[b], seml[b]),
                pltpu.async_copy(tar_hbm.at[pl.ds(base, CHUNK)],
                                 tarb.at[b], seml[b]))

    loads = {0: issue_load(0)}
    stores = {}
    for ci in range(NCHUNKS):
        b = ci % 2
        if ci + 1 < NCHUNKS:
            if ci - 1 in stores:
                stores.pop(ci - 1).wait()
            loads[ci + 1] = issue_load(ci + 1)
        for cp in loads.pop(ci):
            cp.wait()

        @plsc.parallel_loop(0, CHUNK // 16, unroll=8)
        def _(i):
            nv = netb[b, pl.ds(i * 16, 16)]
            tv = tarb[b, pl.ds(i * 16, 16)]
            tm = tv >= 0.5
            sel = jnp.where(tm, nv, jnp.float32(1.0) - nv)
            netb[b, pl.ds(i * 16, 16)] = jnp.where(tm, -sel, sel)
            flat = lane2048 + lax.shift_right_logical(
                plsc.bitcast(sel, jnp.int32), 21)
            plsc.addupdate_scatter(histb, [flat], ones)

        base = wid * EPT + ci * CHUNK
        stores[ci] = pltpu.async_copy(netb.at[b],
                                      u_hbm.at[pl.ds(base, CHUNK)], semu[b])
    for ci in list(stores):
        stores.pop(ci).wait()
    _collapse(histb, collb, 2048)
    pltpu.sync_copy(collb, h1_hbm.at[pl.ds(wid * 2048, 2048)])


def _refine_pass(u_hbm, ub, histb, wid, nbins, bin_fn, seml):
    lane_n = lax.iota(jnp.int32, 16) * nbins
    ones = jnp.ones((16,), jnp.int32)

    def issue_load(ci):
        b = ci % 2
        base = wid * EPT + ci * CHUNK
        return pltpu.async_copy(u_hbm.at[pl.ds(base, CHUNK)], ub.at[b],
                                seml[b])

    loads = {0: issue_load(0)}
    for ci in range(NCHUNKS):
        b = ci % 2
        if ci + 1 < NCHUNKS:
            loads[ci + 1] = issue_load(ci + 1)
        loads.pop(ci).wait()

        @plsc.parallel_loop(0, CHUNK // 16, unroll=8)
        def _(i):
            u = jnp.bitwise_and(
                plsc.bitcast(ub[b, pl.ds(i * 16, 16)], jnp.int32), MAGN)
            bn, pm = bin_fn(u)
            plsc.addupdate_scatter(histb, [lane_n + bn], ones, mask=pm)


@functools.partial(
    pl.kernel,
    out_type=[jax.ShapeDtypeStruct((NW * 2048,), jnp.int32),
              jax.ShapeDtypeStruct((16,), jnp.int32)],
    mesh=_MESH,
    compiler_params=_SC_PARAMS,
    scratch_types=[
        pltpu.VMEM((2, CHUNK), jnp.float32),
        pltpu.VMEM((LANES * 2048,), jnp.int32),
        pltpu.VMEM((2048,), jnp.int32),
        pltpu.VMEM((8 * 2048,), jnp.int32),
        pltpu.VMEM((2048,), jnp.int32),
        pltpu.VMEM((16,), jnp.int32),
        pltpu.SemaphoreType.DMA,
        pltpu.SemaphoreType.DMA,
    ],
)
def _sc_pass2(u_hbm, h1_hbm, out_hbm, meta_hbm,
              ub, histb, collb, blkb, mergedb, metab, seml0, seml1):
    wid = lax.axis_index("s") * NC + lax.axis_index("c")
    _merge_parts(h1_hbm, blkb, mergedb, 2048)
    b1, c1 = _scan_merged(mergedb, 2048, jnp.int32(K_TOP))
    _zero_hist(histb, LANES * 2048)

    def bin_fn(u):
        pm = lax.shift_right_logical(u, 21) == b1
        return jnp.bitwise_and(lax.shift_right_logical(u, 10), 0x7FF), pm

    _refine_pass(u_hbm, ub, histb, wid, 2048, bin_fn, [seml0, seml1])
    _collapse(histb, collb, 2048)
    pltpu.sync_copy(collb, out_hbm.at[pl.ds(wid * 2048, 2048)])
    _emit_meta(metab, meta_hbm, wid, [b1, c1])


@functools.partial(
    pl.kernel,
    out_type=[jax.ShapeDtypeStruct((NW * 1024,), jnp.int32),
              jax.ShapeDtypeStruct((16,), jnp.int32)],
    mesh=_MESH,
    compiler_params=_SC_PARAMS,
    scratch_types=[
        pltpu.VMEM((2, CHUNK), jnp.float32),
        pltpu.VMEM((LANES * 1024,), jnp.int32),
        pltpu.VMEM((1024,), jnp.int32),
        pltpu.VMEM((8 * 2048,), jnp.int32),
        pltpu.VMEM((2048,), jnp.int32),
        pltpu.VMEM((16,), jnp.int32),
        pltpu.SemaphoreType.DMA,
        pltpu.SemaphoreType.DMA,
    ],
)
def _sc_pass3(u_hbm, h2_hbm, meta2_hbm, out_hbm, meta_hbm,
              ub, histb, collb, blkb, mergedb, metab, seml0, seml1):
    wid = lax.axis_index("s") * NC + lax.axis_index("c")
    b1 = _read_meta(metab, meta2_hbm, 0)
    c1 = _read_meta(metab, meta2_hbm, 1)
    _merge_parts(h2_hbm, blkb, mergedb, 2048)
    b2, _ = _scan_merged(mergedb, 2048, jnp.int32(K_TOP) - c1)
    pref21 = jnp.bitwise_or(lax.shift_left(b1, 11), b2)
    _zero_hist(histb, LANES * 1024)

    def bin_fn(u):
        pm = lax.shift_right_logical(u, 10) == pref21
        return jnp.bitwise_and(u, 0x3FF), pm

    _refine_pass(u_hbm, ub, histb, wid, 1024, bin_fn, [seml0, seml1])
    _collapse(histb, collb, 1024)
    pltpu.sync_copy(collb, out_hbm.at[pl.ds(wid * 1024, 1024)])
    _emit_meta(metab, meta_hbm, wid, [pref21])


_TC_GRID = 8
_TC_ROWS = 4096 // _TC_GRID


def _tca_body(u_ref, h2_ref, meta_ref, oi_ref, os_ref, ol_ref, oc_ref,
              acc_i, acc_s, acc_log, acc_cnt, psm):
    step = pl.program_id(0)

    @pl.when(step == 0)
    def _():
        zeros = jnp.zeros((8, 128), jnp.float32)
        acc_i[...] = zeros
        acc_s[...] = zeros
        acc_log[...] = zeros
        acc_cnt[...] = zeros
        h = h2_ref[...].astype(jnp.float32)
        hm = jnp.sum(h, axis=0).reshape(16, 128)
        ri = lax.broadcasted_iota(jnp.int32, (128, 128), 0)
        ci = lax.broadcasted_iota(jnp.int32, (128, 128), 1)
        upper_incl = (ri <= ci).astype(jnp.float32)
        rowcum = jnp.dot(hm, upper_incl, preferred_element_type=jnp.float32)
        rowsum = rowcum[:, 127:128]
        r16 = lax.broadcasted_iota(jnp.int32, (16, 16), 0)
        c16 = lax.broadcasted_iota(jnp.int32, (16, 16), 1)
        strict_lower = (c16 < r16).astype(jnp.float32)
        roff = jnp.dot(strict_lower, rowsum, preferred_element_type=jnp.float32)
        incl = rowcum + roff
        kk = (jnp.int32(K_TOP) - meta_ref[1]).astype(jnp.float32)
        b2 = jnp.sum((incl < kk).astype(jnp.int32))
        pref21 = lax.shift_left(meta_ref[0], 11) + b2
        psm[0] = lax.shift_left(pref21, 10)

    p_lo = psm[0]
    usf = u_ref[...]
    tmask = usf < 0
    sel = jnp.abs(usf)
    u = lax.bitcast_convert_type(sel, jnp.int32)
    below = u < p_lo

    def red(x):
        return jnp.sum(x.reshape(_TC_ROWS, 8, 128), axis=0)

    lgx = jnp.where(below, sel + SMOOTH, jnp.float32(1.0))
    lgy = lgx.reshape(_TC_ROWS // 8, 8, 1024)
    prod = lgy[:, 0, :]
    for j in range(1, 8):
        prod = prod * lgy[:, j, :]
    lg = jnp.log(prod)

    acc_i[...] += red(jnp.where(tmask, sel, jnp.float32(0.0)))
    acc_s[...] += red(jnp.where(tmask, sel, -sel))
    acc_log[...] += jnp.sum(lg.reshape(_TC_ROWS // 8, 8, 128), axis=0)
    acc_cnt[...] += red(below.astype(jnp.float32))

    @pl.when(step == _TC_GRID - 1)
    def _():
        oi_ref[...] = acc_i[...]
        os_ref[...] = acc_s[...]
        ol_ref[...] = acc_log[...]
        oc_ref[...] = acc_cnt[...]


def _tc_stream(u2d, h2, meta2):
    shp = jax.ShapeDtypeStruct((8, 128), jnp.float32)
    acc = pltpu.VMEM((8, 128), jnp.float32)
    return pl.pallas_call(
        _tca_body,
        grid=(_TC_GRID,),
        in_specs=[
            pl.BlockSpec((_TC_ROWS, 1024), lambda i: (i, 0)),
            pl.BlockSpec((NW, 2048), lambda i: (0, 0)),
            pl.BlockSpec(memory_space=pltpu.SMEM),
        ],
        out_specs=[pl.BlockSpec((8, 128), lambda i: (0, 0))] * 4,
        out_shape=[shp] * 4,
        scratch_shapes=[acc] * 4 + [pltpu.SMEM((2,), jnp.int32)],
    )(u2d, h2, meta2)


def _tcb_body(h3_ref, meta_ref, pi_ref, ps_ref, plog_ref, pcnt_ref, out_ref):
    h = h3_ref[...].astype(jnp.float32)
    hm = jnp.sum(h, axis=0).reshape(8, 128)
    ri = lax.broadcasted_iota(jnp.int32, (128, 128), 0)
    ci = lax.broadcasted_iota(jnp.int32, (128, 128), 1)
    upper_incl = (ri <= ci).astype(jnp.float32)
    rowcum = jnp.dot(hm, upper_incl, preferred_element_type=jnp.float32)
    rowsum = rowcum[:, 127:128]
    r8 = lax.broadcasted_iota(jnp.int32, (8, 8), 0)
    c8 = lax.broadcasted_iota(jnp.int32, (8, 8), 1)
    strict_lower = (c8 < r8).astype(jnp.float32)
    roff = jnp.dot(strict_lower, rowsum, preferred_element_type=jnp.float32)
    excl = rowcum + roff - hm

    c_sub = jnp.sum(pcnt_ref[...])
    slog_sub = jnp.sum(plog_ref[...])
    kk = jnp.float32(K_TOP) - c_sub
    sc = jnp.clip(kk - excl, jnp.float32(0.0), hm)
    p_lo = lax.shift_left(meta_ref[0], 10)
    bidx = (lax.broadcasted_iota(jnp.int32, (8, 128), 0) * 128
            + lax.broadcasted_iota(jnp.int32, (8, 128), 1))
    vals = lax.bitcast_convert_type(p_lo + bidx, jnp.float32)
    lg3 = jnp.log(vals + SMOOTH)
    slog_tail = jnp.sum(sc * lg3)

    inter = jnp.sum(pi_ref[...])
    union = jnp.float32(N_ELEMS) + jnp.sum(ps_ref[...])
    dc = jnp.float32(1.0) - jnp.float32(2.0) * (inter + EPSILON) / (union + EPSILON)
    ce = -(slog_sub + slog_tail) / jnp.float32(K_TOP)
    res = ce + dc
    col = lax.broadcasted_iota(jnp.int32, (8, 128), 1)
    row = lax.broadcasted_iota(jnp.int32, (8, 128), 0)
    out = jnp.where((row == 0) & (col == 0), res, jnp.float32(0.0))
    out = jnp.where((row == 0) & (col == 1), ce, out)
    out = jnp.where((row == 0) & (col == 2), dc, out)
    out_ref[...] = out


def _tc_final(h3, meta3, pi, ps, plog, pcnt):
    blk = pl.BlockSpec((8, 128), lambda: (0, 0))
    return pl.pallas_call(
        _tcb_body,
        in_specs=[
            pl.BlockSpec((NW, 1024), lambda: (0, 0)),
            pl.BlockSpec(memory_space=pltpu.SMEM),
            blk, blk, blk, blk,
        ],
        out_specs=pl.BlockSpec((8, 128), lambda: (0, 0)),
        out_shape=jax.ShapeDtypeStruct((8, 128), jnp.float32),
    )(h3, meta3, pi, ps, plog, pcnt)


def kernel(net_output, target):
    net = net_output.reshape(-1)
    tar = target.reshape(-1)
    usf, h1 = _sc_pass1(net, tar)
    h2, meta2 = _sc_pass2(usf, h1)
    h3, meta3 = _sc_pass3(usf, h2, meta2)
    pi, ps, plog, pcnt = _tc_stream(usf.reshape(4096, 1024),
                                    h2.reshape(NW, 2048), meta2)
    out = _tc_final(h3.reshape(NW, 1024), meta3, pi, ps, plog, pcnt)
    return out[0, 0], out[0, 1], out[0, 2]


# speedup vs baseline: 1.0242x; 1.0242x over previous
import functools

import jax
import jax.numpy as jnp
import numpy as np
from jax import lax
from jax.experimental import pallas as pl
from jax.experimental.pallas import tpu as pltpu
from jax.experimental.pallas import tpu_sc as plsc

N_ELEMS = 2 * 128 * 128 * 128
K_TOP = int(N_ELEMS * 10 / 100)
SMOOTH = np.float32(1e-4)
EPSILON = np.float32(1e-5)
MAGN = 0x7FFFFFFF

NC, NS, LANES = 2, 16, 16
NW = NC * NS
EPT = N_ELEMS // NW
CHUNK = 8192
NCHUNKS = EPT // CHUNK

_MESH = plsc.VectorSubcoreMesh(
    core_axis_name="c", subcore_axis_name="s", num_cores=NC, num_subcores=NS)
_SC_PARAMS = pltpu.CompilerParams(needs_layout_passes=False)


def _merge_parts(h_hbm, blkb, mergedb, nbins):
    zeros = jnp.zeros((16,), jnp.int32)

    @plsc.parallel_loop(0, nbins // 16, unroll=8)
    def _(i):
        mergedb[pl.ds(i * 16, 16)] = zeros

    for blk in range(NW // 8):
        pltpu.sync_copy(h_hbm.at[pl.ds(blk * 8 * nbins, 8 * nbins)], blkb)

        @plsc.parallel_loop(0, nbins // 16, unroll=4)
        def _(g):
            acc = mergedb[pl.ds(g * 16, 16)]
            for r in range(8):
                acc = acc + blkb[pl.ds(r * nbins + g * 16, 16)]
            mergedb[pl.ds(g * 16, 16)] = acc


def _scan_merged(mergedb, nbins, kk):
    def body(g, carry):
        b_acc, c_acc, run = carry
        acc = mergedb[pl.ds(g * 16, 16)]
        incl = plsc.cumsum(acc) + run
        mask = incl < kk
        b_acc = b_acc + jnp.sum(jnp.where(mask, 1, 0))
        c_acc = c_acc + jnp.sum(jnp.where(mask, acc, 0))
        run = run + jnp.sum(acc)
        return b_acc, c_acc, run

    b, c, _ = lax.fori_loop(0, nbins // 16, body,
                            (jnp.int32(0), jnp.int32(0), jnp.int32(0)))
    return b, c


def _zero_hist(histb, nwords):
    zeros = jnp.zeros((16,), jnp.int32)

    @plsc.parallel_loop(0, nwords // 16, unroll=8)
    def _(i):
        histb[pl.ds(i * 16, 16)] = zeros


def _collapse(hists, collb, nbins):
    @plsc.parallel_loop(0, nbins // 16, unroll=2)
    def _(g):
        acc = jnp.zeros((16,), jnp.int32)
        for hb in hists:
            for l in range(LANES):
                acc = acc + hb[pl.ds(l * nbins + g * 16, 16)]
        collb[pl.ds(g * 16, 16)] = acc


def _emit_meta(metab, meta_hbm, wid, vals):
    idx16 = lax.iota(jnp.int32, 16)
    v = jnp.zeros((16,), jnp.int32)
    for j, s in enumerate(vals):
        v = jnp.where(idx16 == j, s, v)

    @pl.when(wid == 0)
    def _():
        metab[...] = v
        pltpu.sync_copy(metab, meta_hbm)


def _read_meta(metab, meta_hbm, j):
    idx16 = lax.iota(jnp.int32, 16)
    pltpu.sync_copy(meta_hbm, metab)
    v = metab[...]
    return jnp.sum(jnp.where(idx16 == j, v, 0))


@functools.partial(
    pl.kernel,
    out_type=[jax.ShapeDtypeStruct((N_ELEMS,), jnp.float32),
              jax.ShapeDtypeStruct((NW * 2048,), jnp.int32)],
    mesh=_MESH,
    compiler_params=_SC_PARAMS,
    scratch_types=[
        pltpu.VMEM((2, CHUNK), jnp.float32),
        pltpu.VMEM((2, CHUNK), jnp.float32),
        pltpu.VMEM((LANES * 2048,), jnp.int32),
        pltpu.VMEM((LANES * 2048,), jnp.int32),
        pltpu.VMEM((2048,), jnp.int32),
        pltpu.SemaphoreType.DMA,
        pltpu.SemaphoreType.DMA,
        pltpu.SemaphoreType.DMA,
        pltpu.SemaphoreType.DMA,
    ],
)
def _sc_pass1(net_hbm, tar_hbm, u_hbm, h1_hbm,
              netb, tarb, histb, histb2, collb, seml0, seml1, semu0, semu1):
    wid = lax.axis_index("s") * NC + lax.axis_index("c")
    _zero_hist(histb, LANES * 2048)
    _zero_hist(histb2, LANES * 2048)
    lane2048 = lax.iota(jnp.int32, 16) * 2048
    ones = jnp.ones((16,), jnp.int32)
    seml = [seml0, seml1]
    semu = [semu0, semu1]

    def issue_load(ci):
        b = ci % 2
        base = wid * EPT + ci * CHUNK
        return (pltpu.async_copy(net_hbm.at[pl.ds(base, CHUNK)],
                                 netb.at[b], seml[b]),
                pltpu.async_copy(tar_hbm.at[pl.ds(base, CHUNK)],
                                 tarb.at[b], seml[b]))

    loads = {0: issue_load(0)}
    stores = {}
    for ci in range(NCHUNKS):
        b = ci % 2
        if ci + 1 < NCHUNKS:
            if ci - 1 in stores:
                stores.pop(ci - 1).wait()
            loads[ci + 1] = issue_load(ci + 1)
        for cp in loads.pop(ci):
            cp.wait()

        @plsc.parallel_loop(0, CHUNK // 32, unroll=4)
        def _(i):
            for j, hb in ((0, histb), (1, histb2)):
                off = (i * 2 + j) * 16
                nv = netb[b, pl.ds(off, 16)]
                tv = tarb[b, pl.ds(off, 16)]
                tm = tv >= 0.5
                sel = jnp.where(tm, nv, jnp.float32(1.0) - nv)
                netb[b, pl.ds(off, 16)] = jnp.where(tm, -sel, sel)
                flat = lane2048 + lax.shift_right_logical(
                    plsc.bitcast(sel, jnp.int32), 21)
                plsc.addupdate_scatter(hb, [flat], ones)

        base = wid * EPT + ci * CHUNK
        stores[ci] = pltpu.async_copy(netb.at[b],
                                      u_hbm.at[pl.ds(base, CHUNK)], semu[b])
    for ci in list(stores):
        stores.pop(ci).wait()
    _collapse([histb, histb2], collb, 2048)
    pltpu.sync_copy(collb, h1_hbm.at[pl.ds(wid * 2048, 2048)])


def _refine_pass(u_hbm, ub, hists, wid, nbins, bin_fn, seml):
    lane_n = lax.iota(jnp.int32, 16) * nbins
    ones = jnp.ones((16,), jnp.int32)

    def issue_load(ci):
        b = ci % 2
        base = wid * EPT + ci * CHUNK
        return pltpu.async_copy(u_hbm.at[pl.ds(base, CHUNK)], ub.at[b],
                                seml[b])

    loads = {0: issue_load(0)}
    for ci in range(NCHUNKS):
        b = ci % 2
        if ci + 1 < NCHUNKS:
            loads[ci + 1] = issue_load(ci + 1)
        loads.pop(ci).wait()

        @plsc.parallel_loop(0, CHUNK // 32, unroll=4)
        def _(i):
            for j, hb in enumerate(hists):
                off = (i * 2 + j) * 16
                u = jnp.bitwise_and(
                    plsc.bitcast(ub[b, pl.ds(off, 16)], jnp.int32), MAGN)
                bn, pm = bin_fn(u)
                plsc.addupdate_scatter(hb, [lane_n + bn], ones, mask=pm)


@functools.partial(
    pl.kernel,
    out_type=[jax.ShapeDtypeStruct((NW * 2048,), jnp.int32),
              jax.ShapeDtypeStruct((16,), jnp.int32)],
    mesh=_MESH,
    compiler_params=_SC_PARAMS,
    scratch_types=[
        pltpu.VMEM((2, CHUNK), jnp.float32),
        pltpu.VMEM((LANES * 2048,), jnp.int32),
        pltpu.VMEM((LANES * 2048,), jnp.int32),
        pltpu.VMEM((2048,), jnp.int32),
        pltpu.VMEM((8 * 2048,), jnp.int32),
        pltpu.VMEM((2048,), jnp.int32),
        pltpu.VMEM((16,), jnp.int32),
        pltpu.SemaphoreType.DMA,
        pltpu.SemaphoreType.DMA,
    ],
)
def _sc_pass2(u_hbm, h1_hbm, out_hbm, meta_hbm,
              ub, histb, histb2, collb, blkb, mergedb, metab, seml0, seml1):
    wid = lax.axis_index("s") * NC + lax.axis_index("c")
    _merge_parts(h1_hbm, blkb, mergedb, 2048)
    b1, c1 = _scan_merged(mergedb, 2048, jnp.int32(K_TOP))
    _zero_hist(histb, LANES * 2048)
    _zero_hist(histb2, LANES * 2048)

    def bin_fn(u):
        pm = lax.shift_right_logical(u, 21) == b1
        return jnp.bitwise_and(lax.shift_right_logical(u, 10), 0x7FF), pm

    _refine_pass(u_hbm, ub, [histb, histb2], wid, 2048, bin_fn,
                 [seml0, seml1])
    _collapse([histb, histb2], collb, 2048)
    pltpu.sync_copy(collb, out_hbm.at[pl.ds(wid * 2048, 2048)])
    _emit_meta(metab, meta_hbm, wid, [b1, c1])


@functools.partial(
    pl.kernel,
    out_type=[jax.ShapeDtypeStruct((NW * 1024,), jnp.int32),
              jax.ShapeDtypeStruct((16,), jnp.int32)],
    mesh=_MESH,
    compiler_params=_SC_PARAMS,
    scratch_types=[
        pltpu.VMEM((2, CHUNK), jnp.float32),
        pltpu.VMEM((LANES * 1024,), jnp.int32),
        pltpu.VMEM((LANES * 1024,), jnp.int32),
        pltpu.VMEM((1024,), jnp.int32),
        pltpu.VMEM((8 * 2048,), jnp.int32),
        pltpu.VMEM((2048,), jnp.int32),
        pltpu.VMEM((16,), jnp.int32),
        pltpu.SemaphoreType.DMA,
        pltpu.SemaphoreType.DMA,
    ],
)
def _sc_pass3(u_hbm, h2_hbm, meta2_hbm, out_hbm, meta_hbm,
              ub, histb, histb2, collb, blkb, mergedb, metab, seml0, seml1):
    wid = lax.axis_index("s") * NC + lax.axis_index("c")
    b1 = _read_meta(metab, meta2_hbm, 0)
    c1 = _read_meta(metab, meta2_hbm, 1)
    _merge_parts(h2_hbm, blkb, mergedb, 2048)
    b2, _ = _scan_merged(mergedb, 2048, jnp.int32(K_TOP) - c1)
    pref21 = jnp.bitwise_or(lax.shift_left(b1, 11), b2)
    _zero_hist(histb, LANES * 1024)
    _zero_hist(histb2, LANES * 1024)

    def bin_fn(u):
        pm = lax.shift_right_logical(u, 10) == pref21
        return jnp.bitwise_and(u, 0x3FF), pm

    _refine_pass(u_hbm, ub, [histb, histb2], wid, 1024, bin_fn,
                 [seml0, seml1])
    _collapse([histb, histb2], collb, 1024)
    pltpu.sync_copy(collb, out_hbm.at[pl.ds(wid * 1024, 1024)])
    _emit_meta(metab, meta_hbm, wid, [pref21])


_TC_GRID = 8
_TC_ROWS = 4096 // _TC_GRID


def _tca_body(u_ref, h2_ref, meta_ref, oi_ref, os_ref, ol_ref, oc_ref,
              acc_i, acc_s, acc_log, acc_cnt, psm):
    step = pl.program_id(0)

    @pl.when(step == 0)
    def _():
        zeros = jnp.zeros((8, 128), jnp.float32)
        acc_i[...] = zeros
        acc_s[...] = zeros
        acc_log[...] = zeros
        acc_cnt[...] = zeros
        h = h2_ref[...].astype(jnp.float32)
        hm = jnp.sum(h, axis=0).reshape(16, 128)
        ri = lax.broadcasted_iota(jnp.int32, (128, 128), 0)
        ci = lax.broadcasted_iota(jnp.int32, (128, 128), 1)
        upper_incl = (ri <= ci).astype(jnp.float32)
        rowcum = jnp.dot(hm, upper_incl, preferred_element_type=jnp.float32)
        rowsum = rowcum[:, 127:128]
        r16 = lax.broadcasted_iota(jnp.int32, (16, 16), 0)
        c16 = lax.broadcasted_iota(jnp.int32, (16, 16), 1)
        strict_lower = (c16 < r16).astype(jnp.float32)
        roff = jnp.dot(strict_lower, rowsum, preferred_element_type=jnp.float32)
        incl = rowcum + roff
        kk = (jnp.int32(K_TOP) - meta_ref[1]).astype(jnp.float32)
        b2 = jnp.sum((incl < kk).astype(jnp.int32))
        pref21 = lax.shift_left(meta_ref[0], 11) + b2
        psm[0] = lax.shift_left(pref21, 10)

    p_lo = psm[0]
    usf = u_ref[...]
    tmask = usf < 0
    sel = jnp.abs(usf)
    u = lax.bitcast_convert_type(sel, jnp.int32)
    below = u < p_lo
    lg = jnp.log(sel + SMOOTH)

    def red(x):
        return jnp.sum(x.reshape(_TC_ROWS, 8, 128), axis=0)

    acc_i[...] += red(jnp.where(tmask, sel, jnp.float32(0.0)))
    acc_s[...] += red(jnp.where(tmask, sel, -sel))
    acc_log[...] += red(jnp.where(below, lg, jnp.float32(0.0)))
    acc_cnt[...] += red(below.astype(jnp.float32))

    @pl.when(step == _TC_GRID - 1)
    def _():
        oi_ref[...] = acc_i[...]
        os_ref[...] = acc_s[...]
        ol_ref[...] = acc_log[...]
        oc_ref[...] = acc_cnt[...]


def _tc_stream(u2d, h2, meta2):
    shp = jax.ShapeDtypeStruct((8, 128), jnp.float32)
    acc = pltpu.VMEM((8, 128), jnp.float32)
    return pl.pallas_call(
        _tca_body,
        grid=(_TC_GRID,),
        in_specs=[
            pl.BlockSpec((_TC_ROWS, 1024), lambda i: (i, 0)),
            pl.BlockSpec((NW, 2048), lambda i: (0, 0)),
            pl.BlockSpec(memory_space=pltpu.SMEM),
        ],
        out_specs=[pl.BlockSpec((8, 128), lambda i: (0, 0))] * 4,
        out_shape=[shp] * 4,
        scratch_shapes=[acc] * 4 + [pltpu.SMEM((2,), jnp.int32)],
    )(u2d, h2, meta2)


def _tcb_body(h3_ref, meta_ref, pi_ref, ps_ref, plog_ref, pcnt_ref, out_ref):
    h = h3_ref[...].astype(jnp.float32)
    hm = jnp.sum(h, axis=0).reshape(8, 128)
    ri = lax.broadcasted_iota(jnp.int32, (128, 128), 0)
    ci = lax.broadcasted_iota(jnp.int32, (128, 128), 1)
    upper_incl = (ri <= ci).astype(jnp.float32)
    rowcum = jnp.dot(hm, upper_incl, preferred_element_type=jnp.float32)
    rowsum = rowcum[:, 127:128]
    r8 = lax.broadcasted_iota(jnp.int32, (8, 8), 0)
    c8 = lax.broadcasted_iota(jnp.int32, (8, 8), 1)
    strict_lower = (c8 < r8).astype(jnp.float32)
    roff = jnp.dot(strict_lower, rowsum, preferred_element_type=jnp.float32)
    excl = rowcum + roff - hm

    c_sub = jnp.sum(pcnt_ref[...])
    slog_sub = jnp.sum(plog_ref[...])
    kk = jnp.float32(K_TOP) - c_sub
    sc = jnp.clip(kk - excl, jnp.float32(0.0), hm)
    p_lo = lax.shift_left(meta_ref[0], 10)
    bidx = (lax.broadcasted_iota(jnp.int32, (8, 128), 0) * 128
            + lax.broadcasted_iota(jnp.int32, (8, 128), 1))
    vals = lax.bitcast_convert_type(p_lo + bidx, jnp.float32)
    lg3 = jnp.log(vals + SMOOTH)
    slog_tail = jnp.sum(sc * lg3)

    inter = jnp.sum(pi_ref[...])
    union = jnp.float32(N_ELEMS) + jnp.sum(ps_ref[...])
    dc = jnp.float32(1.0) - jnp.float32(2.0) * (inter + EPSILON) / (union + EPSILON)
    ce = -(slog_sub + slog_tail) / jnp.float32(K_TOP)
    res = ce + dc
    col = lax.broadcasted_iota(jnp.int32, (8, 128), 1)
    row = lax.broadcasted_iota(jnp.int32, (8, 128), 0)
    out = jnp.where((row == 0) & (col == 0), res, jnp.float32(0.0))
    out = jnp.where((row == 0) & (col == 1), ce, out)
    out = jnp.where((row == 0) & (col == 2), dc, out)
    out_ref[...] = out


def _tc_final(h3, meta3, pi, ps, plog, pcnt):
    blk = pl.BlockSpec((8, 128), lambda: (0, 0))
    return pl.pallas_call(
        _tcb_body,
        in_specs=[
            pl.BlockSpec((NW, 1024), lambda: (0, 0)),
            pl.BlockSpec(memory_space=pltpu.SMEM),
            blk, blk, blk, blk,
        ],
        out_specs=pl.BlockSpec((8, 128), lambda: (0, 0)),
        out_shape=jax.ShapeDtypeStruct((8, 128), jnp.float32),
    )(h3, meta3, pi, ps, plog, pcnt)


def kernel(net_output, target):
    net = net_output.reshape(-1)
    tar = target.reshape(-1)
    usf, h1 = _sc_pass1(net, tar)
    h2, meta2 = _sc_pass2(usf, h1)
    h3, meta3 = _sc_pass3(usf, h2, meta2)
    pi, ps, plog, pcnt = _tc_stream(usf.reshape(4096, 1024),
                                    h2.reshape(NW, 2048), meta2)
    out = _tc_final(h3.reshape(NW, 1024), meta3, pi, ps, plog, pcnt)
    return out[0, 0], out[0, 1], out[0, 2]


# speedup vs baseline: 1.0976x; 1.0717x over previous
import functools

import jax
import jax.numpy as jnp
import numpy as np
from jax import lax
from jax.experimental import pallas as pl
from jax.experimental.pallas import tpu as pltpu
from jax.experimental.pallas import tpu_sc as plsc

N_ELEMS = 2 * 128 * 128 * 128
K_TOP = int(N_ELEMS * 10 / 100)
SMOOTH = np.float32(1e-4)
EPSILON = np.float32(1e-5)
MAGN = 0x7FFFFFFF

NC, NS, LANES = 2, 16, 16
NW = NC * NS
EPT = N_ELEMS // NW
CHUNK = 16384
NCHUNKS = EPT // CHUNK

_MESH = plsc.VectorSubcoreMesh(
    core_axis_name="c", subcore_axis_name="s", num_cores=NC, num_subcores=NS)
_SC_PARAMS = pltpu.CompilerParams(needs_layout_passes=False)


def _merge_parts(h_hbm, blkb, mergedb, nbins):
    zeros = jnp.zeros((16,), jnp.int32)

    @plsc.parallel_loop(0, nbins // 16, unroll=8)
    def _(i):
        mergedb[pl.ds(i * 16, 16)] = zeros

    for blk in range(NW // 8):
        pltpu.sync_copy(h_hbm.at[pl.ds(blk * 8 * nbins, 8 * nbins)], blkb)

        @plsc.parallel_loop(0, nbins // 16, unroll=4)
        def _(g):
            acc = mergedb[pl.ds(g * 16, 16)]
            for r in range(8):
                acc = acc + blkb[pl.ds(r * nbins + g * 16, 16)]
            mergedb[pl.ds(g * 16, 16)] = acc


def _scan_merged(mergedb, nbins, kk):
    def body(g, carry):
        b_acc, c_acc, run = carry
        acc = mergedb[pl.ds(g * 16, 16)]
        incl = plsc.cumsum(acc) + run
        mask = incl < kk
        b_acc = b_acc + jnp.sum(jnp.where(mask, 1, 0))
        c_acc = c_acc + jnp.sum(jnp.where(mask, acc, 0))
        run = run + jnp.sum(acc)
        return b_acc, c_acc, run

    b, c, _ = lax.fori_loop(0, nbins // 16, body,
                            (jnp.int32(0), jnp.int32(0), jnp.int32(0)))
    return b, c


def _zero_hist(histb, nwords):
    zeros = jnp.zeros((16,), jnp.int32)

    @plsc.parallel_loop(0, nwords // 16, unroll=8)
    def _(i):
        histb[pl.ds(i * 16, 16)] = zeros


def _collapse(hists, collb, nbins):
    @plsc.parallel_loop(0, nbins // 16, unroll=2)
    def _(g):
        acc = jnp.zeros((16,), jnp.int32)
        for hb in hists:
            for l in range(LANES):
                acc = acc + hb[pl.ds(l * nbins + g * 16, 16)]
        collb[pl.ds(g * 16, 16)] = acc


def _emit_meta(metab, meta_hbm, wid, vals):
    idx16 = lax.iota(jnp.int32, 16)
    v = jnp.zeros((16,), jnp.int32)
    for j, s in enumerate(vals):
        v = jnp.where(idx16 == j, s, v)

    @pl.when(wid == 0)
    def _():
        metab[...] = v
        pltpu.sync_copy(metab, meta_hbm)


def _read_meta(metab, meta_hbm, j):
    idx16 = lax.iota(jnp.int32, 16)
    pltpu.sync_copy(meta_hbm, metab)
    v = metab[...]
    return jnp.sum(jnp.where(idx16 == j, v, 0))


@functools.partial(
    pl.kernel,
    out_type=[jax.ShapeDtypeStruct((N_ELEMS,), jnp.float32),
              jax.ShapeDtypeStruct((NW * 2048,), jnp.int32)],
    mesh=_MESH,
    compiler_params=_SC_PARAMS,
    scratch_types=[
        pltpu.VMEM((2, CHUNK), jnp.float32),
        pltpu.VMEM((2, CHUNK), jnp.float32),
        pltpu.VMEM((LANES * 2048,), jnp.int32),
        pltpu.VMEM((2048,), jnp.int32),
        pltpu.SemaphoreType.DMA,
        pltpu.SemaphoreType.DMA,
        pltpu.SemaphoreType.DMA,
        pltpu.SemaphoreType.DMA,
    ],
)
def _sc_pass1(net_hbm, tar_hbm, u_hbm, h1_hbm,
              netb, tarb, histb, collb, seml0, seml1, semu0, semu1):
    wid = lax.axis_index("s") * NC + lax.axis_index("c")
    _zero_hist(histb, LANES * 2048)
    lane2048 = lax.iota(jnp.int32, 16) * 2048
    ones = jnp.ones((16,), jnp.int32)
    seml = [seml0, seml1]
    semu = [semu0, semu1]

    def issue_load(ci):
        b = ci % 2
        base = wid * EPT + ci * CHUNK
        return (pltpu.async_copy(net_hbm.at[pl.ds(base, CHUNK)],
                                 netb.at[b], seml[b]),
                pltpu.async_copy(tar_hbm.at[pl.ds(base, CHUNK)],
                                 tarb.at[b], seml[b]))

    loads = {0: issue_load(0)}
    stores = {}
    for ci in range(NCHUNKS):
        b = ci % 2
        if ci + 1 < NCHUNKS:
            if ci - 1 in stores:
                stores.pop(ci - 1).wait()
            loads[ci + 1] = issue_load(ci + 1)
        for cp in loads.pop(ci):
            cp.wait()

        @plsc.parallel_loop(0, CHUNK // 16, unroll=8)
        def _(i):
            nv = netb[b, pl.ds(i * 16, 16)]
            tv = tarb[b, pl.ds(i * 16, 16)]
            tm = tv >= 0.5
            sel = jnp.where(tm, nv, jnp.float32(1.0) - nv)
            netb[b, pl.ds(i * 16, 16)] = jnp.where(tm, -sel, sel)
            flat = lane2048 + lax.shift_right_logical(
                plsc.bitcast(sel, jnp.int32), 21)
            plsc.addupdate_scatter(histb, [flat], ones)

        base = wid * EPT + ci * CHUNK
        stores[ci] = pltpu.async_copy(netb.at[b],
                                      u_hbm.at[pl.ds(base, CHUNK)], semu[b])
    for ci in list(stores):
        stores.pop(ci).wait()
    _collapse([histb], collb, 2048)
    pltpu.sync_copy(collb, h1_hbm.at[pl.ds(wid * 2048, 2048)])


def _refine_pass(u_hbm, ub, hists, wid, nbins, bin_fn, seml):
    lane_n = lax.iota(jnp.int32, 16) * nbins
    ones = jnp.ones((16,), jnp.int32)

    def issue_load(ci):
        b = ci % 2
        base = wid * EPT + ci * CHUNK
        return pltpu.async_copy(u_hbm.at[pl.ds(base, CHUNK)], ub.at[b],
                                seml[b])

    loads = {0: issue_load(0)}
    for ci in range(NCHUNKS):
        b = ci % 2
        if ci + 1 < NCHUNKS:
            loads[ci + 1] = issue_load(ci + 1)
        loads.pop(ci).wait()

        @plsc.parallel_loop(0, CHUNK // 16, unroll=8)
        def _(i):
            u = jnp.bitwise_and(
                plsc.bitcast(ub[b, pl.ds(i * 16, 16)], jnp.int32), MAGN)
            bn, pm = bin_fn(u)
            plsc.addupdate_scatter(hists[0], [lane_n + bn], ones, mask=pm)


@functools.partial(
    pl.kernel,
    out_type=[jax.ShapeDtypeStruct((NW * 2048,), jnp.int32),
              jax.ShapeDtypeStruct((16,), jnp.int32)],
    mesh=_MESH,
    compiler_params=_SC_PARAMS,
    scratch_types=[
        pltpu.VMEM((2, CHUNK), jnp.float32),
        pltpu.VMEM((LANES * 2048,), jnp.int32),
        pltpu.VMEM((2048,), jnp.int32),
        pltpu.VMEM((8 * 2048,), jnp.int32),
        pltpu.VMEM((2048,), jnp.int32),
        pltpu.VMEM((16,), jnp.int32),
        pltpu.SemaphoreType.DMA,
        pltpu.SemaphoreType.DMA,
    ],
)
def _sc_pass2(u_hbm, h1_hbm, out_hbm, meta_hbm,
              ub, histb, collb, blkb, mergedb, metab, seml0, seml1):
    wid = lax.axis_index("s") * NC + lax.axis_index("c")
    _merge_parts(h1_hbm, blkb, mergedb, 2048)
    b1, c1 = _scan_merged(mergedb, 2048, jnp.int32(K_TOP))
    _zero_hist(histb, LANES * 2048)

    def bin_fn(u):
        pm = lax.shift_right_logical(u, 21) == b1
        return jnp.bitwise_and(lax.shift_right_logical(u, 10), 0x7FF), pm

    _refine_pass(u_hbm, ub, [histb], wid, 2048, bin_fn, [seml0, seml1])
    _collapse([histb], collb, 2048)
    pltpu.sync_copy(collb, out_hbm.at[pl.ds(wid * 2048, 2048)])
    _emit_meta(metab, meta_hbm, wid, [b1, c1])


@functools.partial(
    pl.kernel,
    out_type=[jax.ShapeDtypeStruct((NW * 1024,), jnp.int32),
              jax.ShapeDtypeStruct((16,), jnp.int32)],
    mesh=_MESH,
    compiler_params=_SC_PARAMS,
    scratch_types=[
        pltpu.VMEM((2, CHUNK), jnp.float32),
        pltpu.VMEM((LANES * 1024,), jnp.int32),
        pltpu.VMEM((1024,), jnp.int32),
        pltpu.VMEM((8 * 2048,), jnp.int32),
        pltpu.VMEM((2048,), jnp.int32),
        pltpu.VMEM((16,), jnp.int32),
        pltpu.SemaphoreType.DMA,
        pltpu.SemaphoreType.DMA,
    ],
)
def _sc_pass3(u_hbm, h2_hbm, meta2_hbm, out_hbm, meta_hbm,
              ub, histb, collb, blkb, mergedb, metab, seml0, seml1):
    wid = lax.axis_index("s") * NC + lax.axis_index("c")
    b1 = _read_meta(metab, meta2_hbm, 0)
    c1 = _read_meta(metab, meta2_hbm, 1)
    _merge_parts(h2_hbm, blkb, mergedb, 2048)
    b2, _ = _scan_merged(mergedb, 2048, jnp.int32(K_TOP) - c1)
    pref21 = jnp.bitwise_or(lax.shift_left(b1, 11), b2)
    _zero_hist(histb, LANES * 1024)

    def bin_fn(u):
        pm = lax.shift_right_logical(u, 10) == pref21
        return jnp.bitwise_and(u, 0x3FF), pm

    _refine_pass(u_hbm, ub, [histb], wid, 1024, bin_fn, [seml0, seml1])
    _collapse([histb], collb, 1024)
    pltpu.sync_copy(collb, out_hbm.at[pl.ds(wid * 1024, 1024)])
    _emit_meta(metab, meta_hbm, wid, [pref21])


_TC_GRID = 8
_TC_ROWS = 4096 // _TC_GRID


def _tca_body(u_ref, h2_ref, meta_ref, oi_ref, os_ref, ol_ref, oc_ref,
              acc_i, acc_s, acc_log, acc_cnt, psm):
    step = pl.program_id(0)

    @pl.when(step == 0)
    def _():
        zeros = jnp.zeros((8, 128), jnp.float32)
        acc_i[...] = zeros
        acc_s[...] = zeros
        acc_log[...] = zeros
        acc_cnt[...] = zeros
        h = h2_ref[...].astype(jnp.float32)
        hm = jnp.sum(h, axis=0).reshape(16, 128)
        ri = lax.broadcasted_iota(jnp.int32, (128, 128), 0)
        ci = lax.broadcasted_iota(jnp.int32, (128, 128), 1)
        upper_incl = (ri <= ci).astype(jnp.float32)
        rowcum = jnp.dot(hm, upper_incl, preferred_element_type=jnp.float32)
        rowsum = rowcum[:, 127:128]
        r16 = lax.broadcasted_iota(jnp.int32, (16, 16), 0)
        c16 = lax.broadcasted_iota(jnp.int32, (16, 16), 1)
        strict_lower = (c16 < r16).astype(jnp.float32)
        roff = jnp.dot(strict_lower, rowsum, preferred_element_type=jnp.float32)
        incl = rowcum + roff
        kk = (jnp.int32(K_TOP) - meta_ref[1]).astype(jnp.float32)
        b2 = jnp.sum((incl < kk).astype(jnp.int32))
        pref21 = lax.shift_left(meta_ref[0], 11) + b2
        psm[0] = lax.shift_left(pref21, 10)

    p_lo = psm[0]
    usf = u_ref[...]
    tmask = usf < 0
    sel = jnp.abs(usf)
    u = lax.bitcast_convert_type(sel, jnp.int32)
    below = u < p_lo
    lg = jnp.log(sel + SMOOTH)

    def red(x):
        return jnp.sum(x.reshape(_TC_ROWS, 8, 128), axis=0)

    acc_i[...] += red(jnp.where(tmask, sel, jnp.float32(0.0)))
    acc_s[...] += red(jnp.where(tmask, sel, -sel))
    acc_log[...] += red(jnp.where(below, lg, jnp.float32(0.0)))
    acc_cnt[...] += red(below.astype(jnp.float32))

    @pl.when(step == _TC_GRID - 1)
    def _():
        oi_ref[...] = acc_i[...]
        os_ref[...] = acc_s[...]
        ol_ref[...] = acc_log[...]
        oc_ref[...] = acc_cnt[...]


def _tc_stream(u2d, h2, meta2):
    shp = jax.ShapeDtypeStruct((8, 128), jnp.float32)
    acc = pltpu.VMEM((8, 128), jnp.float32)
    return pl.pallas_call(
        _tca_body,
        grid=(_TC_GRID,),
        in_specs=[
            pl.BlockSpec((_TC_ROWS, 1024), lambda i: (i, 0)),
            pl.BlockSpec((NW, 2048), lambda i: (0, 0)),
            pl.BlockSpec(memory_space=pltpu.SMEM),
        ],
        out_specs=[pl.BlockSpec((8, 128), lambda i: (0, 0))] * 4,
        out_shape=[shp] * 4,
        scratch_shapes=[acc] * 4 + [pltpu.SMEM((2,), jnp.int32)],
    )(u2d, h2, meta2)


def _tcb_body(h3_ref, meta_ref, pi_ref, ps_ref, plog_ref, pcnt_ref, out_ref):
    h = h3_ref[...].astype(jnp.float32)
    hm = jnp.sum(h, axis=0).reshape(8, 128)
    ri = lax.broadcasted_iota(jnp.int32, (128, 128), 0)
    ci = lax.broadcasted_iota(jnp.int32, (128, 128), 1)
    upper_incl = (ri <= ci).astype(jnp.float32)
    rowcum = jnp.dot(hm, upper_incl, preferred_element_type=jnp.float32)
    rowsum = rowcum[:, 127:128]
    r8 = lax.broadcasted_iota(jnp.int32, (8, 8), 0)
    c8 = lax.broadcasted_iota(jnp.int32, (8, 8), 1)
    strict_lower = (c8 < r8).astype(jnp.float32)
    roff = jnp.dot(strict_lower, rowsum, preferred_element_type=jnp.float32)
    excl = rowcum + roff - hm

    c_sub = jnp.sum(pcnt_ref[...])
    slog_sub = jnp.sum(plog_ref[...])
    kk = jnp.float32(K_TOP) - c_sub
    sc = jnp.clip(kk - excl, jnp.float32(0.0), hm)
    p_lo = lax.shift_left(meta_ref[0], 10)
    bidx = (lax.broadcasted_iota(jnp.int32, (8, 128), 0) * 128
            + lax.broadcasted_iota(jnp.int32, (8, 128), 1))
    vals = lax.bitcast_convert_type(p_lo + bidx, jnp.float32)
    lg3 = jnp.log(vals + SMOOTH)
    slog_tail = jnp.sum(sc * lg3)

    inter = jnp.sum(pi_ref[...])
    union = jnp.float32(N_ELEMS) + jnp.sum(ps_ref[...])
    dc = jnp.float32(1.0) - jnp.float32(2.0) * (inter + EPSILON) / (union + EPSILON)
    ce = -(slog_sub + slog_tail) / jnp.float32(K_TOP)
    res = ce + dc
    col = lax.broadcasted_iota(jnp.int32, (8, 128), 1)
    row = lax.broadcasted_iota(jnp.int32, (8, 128), 0)
    out = jnp.where((row == 0) & (col == 0), res, jnp.float32(0.0))
    out = jnp.where((row == 0) & (col == 1), ce, out)
    out = jnp.where((row == 0) & (col == 2), dc, out)
    out_ref[...] = out


def _tc_final(h3, meta3, pi, ps, plog, pcnt):
    blk = pl.BlockSpec((8, 128), lambda: (0, 0))
    return pl.pallas_call(
        _tcb_body,
        in_specs=[
            pl.BlockSpec((NW, 1024), lambda: (0, 0)),
            pl.BlockSpec(memory_space=pltpu.SMEM),
            blk, blk, blk, blk,
        ],
        out_specs=pl.BlockSpec((8, 128), lambda: (0, 0)),
        out_shape=jax.ShapeDtypeStruct((8, 128), jnp.float32),
    )(h3, meta3, pi, ps, plog, pcnt)


def kernel(net_output, target):
    net = net_output.reshape(-1)
    tar = target.reshape(-1)
    usf, h1 = _sc_pass1(net, tar)
    h2, meta2 = _sc_pass2(usf, h1)
    h3, meta3 = _sc_pass3(usf, h2, meta2)
    pi, ps, plog, pcnt = _tc_stream(usf.reshape(4096, 1024),
                                    h2.reshape(NW, 2048), meta2)
    out = _tc_final(h3.reshape(NW, 1024), meta3, pi, ps, plog, pcnt)
    return out[0, 0], out[0, 1], out[0, 2]
